# Initial kernel scaffold; baseline (speedup 1.0000x reference)
import functools

import jax
import jax.numpy as jnp
from jax import lax
from jax.experimental import pallas as pl
from jax.experimental.pallas import tpu as pltpu
from jax.experimental.pallas import tpu_sc as plsc

N = 10000
E = 320000
D = 128

NC = 2
NS = 16
NW = NC * NS
CHUNK = 128
CHUNKS = E // CHUNK
FULL = CHUNKS // NW
TAIL = CHUNKS - FULL * NW
RPT = N // NS
CNTW = 16


def _sc_agg(x, src, dst, ones, zd, zc, with_cnt):
    mesh = plsc.VectorSubcoreMesh(core_axis_name="c", subcore_axis_name="s")
    out_type = [jax.ShapeDtypeStruct((NC, N, D), jnp.float32)]
    if with_cnt:
        out_type.append(jax.ShapeDtypeStruct((NC, N, CNTW), jnp.float32))
    scratch = [
        pltpu.VMEM((CHUNK,), jnp.int32),
        pltpu.VMEM((CHUNK,), jnp.int32),
        pltpu.VMEM((CHUNK, D), jnp.float32),
        pltpu.VMEM((CHUNK, CNTW), jnp.float32),
        pltpu.VMEM_SHARED((N, D), jnp.float32),
        pltpu.VMEM_SHARED((N, CNTW), jnp.float32),
        pltpu.SemaphoreType.DMA,
    ]

    def body(x_hbm, src_hbm, dst_hbm, ones_hbm, zd_hbm, zc_hbm,
             agg_out, *rest):
        if with_cnt:
            cnt_out = rest[0]
            rest = rest[1:]
        srcv, dstv, rowsv, onesv, agg_sh, cnt_sh, sem = rest
        cid = lax.axis_index("c")
        sid = lax.axis_index("s")
        wid = sid * NC + cid

        pltpu.sync_copy(zd_hbm, agg_sh.at[pl.ds(sid * RPT, RPT)])
        if with_cnt:
            pltpu.sync_copy(zc_hbm, cnt_sh.at[pl.ds(sid * RPT, RPT)])
            pltpu.sync_copy(ones_hbm, onesv)
        plsc.subcore_barrier()

        def do_chunk(c):
            base = pl.multiple_of(c * CHUNK, CHUNK)
            pltpu.sync_copy(src_hbm.at[pl.ds(base, CHUNK)], srcv)
            pltpu.sync_copy(dst_hbm.at[pl.ds(base, CHUNK)], dstv)
            pltpu.async_copy(x_hbm.at[srcv], rowsv, sem).wait()
            pltpu.sync_copy(rowsv, agg_sh.at[dstv], add=True)
            if with_cnt:
                pltpu.sync_copy(onesv, cnt_sh.at[dstv], add=True)

        @pl.loop(0, FULL)
        def _(k):
            do_chunk(k * NW + wid)

        @pl.when(wid < TAIL)
        def _():
            do_chunk(FULL * NW + wid)

        plsc.subcore_barrier()
        pltpu.sync_copy(agg_sh.at[pl.ds(sid * RPT, RPT)],
                        agg_out.at[cid, pl.ds(sid * RPT, RPT)])
        if with_cnt:
            pltpu.sync_copy(cnt_sh.at[pl.ds(sid * RPT, RPT)],
                            cnt_out.at[cid, pl.ds(sid * RPT, RPT)])

    kern = pl.kernel(body, out_type=out_type, mesh=mesh,
                     scratch_types=scratch)
    return kern(x, src, dst, ones, zd, zc)


BLK = 400


def _tc_dense(aggp, cntp, x, Wl, Wr, b, a):
    def body(aggp_ref, cntp_ref, x_ref, wl_ref, wr_ref, b_ref, a_ref, o_ref):
        s = aggp_ref[0] + aggp_ref[1]
        c = cntp_ref[0, :, 0:1] + cntp_ref[1, :, 0:1]
        agg = s / jnp.maximum(c, 1.0)
        h = (jnp.dot(agg, wl_ref[...], preferred_element_type=jnp.float32)
             + jnp.dot(x_ref[...], wr_ref[...], preferred_element_type=jnp.float32)
             + b_ref[...])
        o_ref[...] = jnp.where(h >= 0, h, a_ref[...] * h)

    return pl.pallas_call(
        body,
        grid=(N // BLK,),
        in_specs=[
            pl.BlockSpec((NC, BLK, D), lambda i: (0, i, 0)),
            pl.BlockSpec((NC, BLK, CNTW), lambda i: (0, i, 0)),
            pl.BlockSpec((BLK, D), lambda i: (i, 0)),
            pl.BlockSpec((D, D), lambda i: (0, 0)),
            pl.BlockSpec((D, D), lambda i: (0, 0)),
            pl.BlockSpec((1, D), lambda i: (0, 0)),
            pl.BlockSpec((1, D), lambda i: (0, 0)),
        ],
        out_specs=pl.BlockSpec((BLK, D), lambda i: (i, 0)),
        out_shape=jax.ShapeDtypeStruct((N, D), jnp.float32),
    )(aggp, cntp, x, Wl, Wr, b.reshape(1, D), a.reshape(1, D))


def kernel(x, edge_index, Wl1, Wr1, b1, a1, Wl2, Wr2, b2, a2):
    src = edge_index[0]
    dst = edge_index[1]
    ones = jnp.ones((CHUNK, CNTW), jnp.float32)
    zd = jnp.zeros((RPT, D), jnp.float32)
    zc = jnp.zeros((RPT, CNTW), jnp.float32)

    agg1p, cntp = _sc_agg(x, src, dst, ones, zd, zc, with_cnt=True)
    h = _tc_dense(agg1p, cntp, x, Wl1, Wr1, b1, a1)
    agg2p = _sc_agg(h, src, dst, ones, zd, zc, with_cnt=False)
    return _tc_dense(agg2p, cntp, h, Wl2, Wr2, b2, a2)


# speedup vs baseline: 6.1726x; 6.1726x over previous
import dataclasses
import functools

import jax
import jax.numpy as jnp
from jax import lax
from jax.experimental import pallas as pl
from jax.experimental.pallas import tpu as pltpu
from jax.experimental.pallas import tpu_sc as plsc

N = 10000
E = 320000
D = 128

NC = 2
NS = 16
NW = NC * NS
CHUNK = 128
CHUNKS = E // CHUNK
FULL = CHUNKS // NW
TAIL = CHUNKS - FULL * NW
RC = 80
NRCH = N // RC
RFULL = NRCH // NS
RTAIL = NRCH - RFULL * NS
CNTW = 16
NP = 10240
CROWS = NP // 8
CPT = CROWS // NS
L = 16


def _sc_agg(x, src, dst, zd, with_cnt):
    mesh = plsc.VectorSubcoreMesh(core_axis_name="c", subcore_axis_name="s")
    NR = N + CROWS if with_cnt else N
    out_type = jax.ShapeDtypeStruct((NC, NR, D), jnp.float32)
    scratch = [
        pltpu.VMEM((CHUNK,), jnp.int32),
        pltpu.VMEM((CHUNK,), jnp.int32),
        pltpu.VMEM((CHUNK,), jnp.int32),
        pltpu.VMEM((CHUNK, D), jnp.float32),
        pltpu.VMEM((CHUNK, D), jnp.float32),
        pltpu.VMEM_SHARED((N, D), jnp.float32),
        pltpu.VMEM_SHARED((CROWS, D), jnp.float32),
        pltpu.SemaphoreType.DMA,
    ]

    def body(x_hbm, src_hbm, dst_hbm, zd_hbm, agg_out, *rest):
        srcv, dstv, ridxv, rowsv, crowsv, agg_sh, cnt_sh, sem = rest
        cid = lax.axis_index("c")
        sid = lax.axis_index("s")
        wid = sid * NC + cid

        def row_chunks(fn):
            @pl.loop(0, RFULL)
            def _(k):
                fn(pl.multiple_of((k * NS + sid) * RC, 8))

            @pl.when(sid < RTAIL)
            def _():
                fn(pl.multiple_of((RFULL * NS + sid) * RC, 8))

        row_chunks(lambda off: pltpu.sync_copy(zd_hbm, agg_sh.at[pl.ds(off, RC)]))
        if with_cnt:
            pltpu.sync_copy(
                zd_hbm, cnt_sh.at[pl.ds(pl.multiple_of(sid * CPT, 8), CPT)])

            zv = jnp.zeros((L,), jnp.float32)

            @pl.loop(0, CHUNK)
            def _(r):
                for j in range(D // L):
                    crowsv[r, pl.ds(j * L, L)] = zv
        plsc.subcore_barrier()

        lanes = lax.iota(jnp.int32, L)
        onev = jnp.full((L,), 1.0, jnp.float32)
        zerov = jnp.zeros((L,), jnp.float32)

        def do_chunk(c):
            base = pl.multiple_of(c * CHUNK, CHUNK)
            pltpu.sync_copy(src_hbm.at[pl.ds(base, CHUNK)], srcv)
            pltpu.sync_copy(dst_hbm.at[pl.ds(base, CHUNK)], dstv)
            cp = pltpu.async_copy(x_hbm.at[srcv], rowsv, sem)
            if with_cnt:
                for j in range(CHUNK // L):
                    d = dstv[pl.ds(j * L, L)]
                    ridxv[pl.ds(j * L, L)] = lax.shift_right_logical(d, 3)
                    cols = (d & 7) * L
                    rows = j * L + lanes
                    plsc.store_scatter(crowsv, [rows, cols], onev)
                pltpu.sync_copy(crowsv, cnt_sh.at[ridxv], add=True)
                for j in range(CHUNK // L):
                    d = dstv[pl.ds(j * L, L)]
                    cols = (d & 7) * L
                    rows = j * L + lanes
                    plsc.store_scatter(crowsv, [rows, cols], zerov)
            cp.wait()
            pltpu.sync_copy(rowsv, agg_sh.at[dstv], add=True)

        @pl.loop(0, FULL)
        def _(k):
            do_chunk(k * NW + wid)

        @pl.when(wid < TAIL)
        def _():
            do_chunk(FULL * NW + wid)

        plsc.subcore_barrier()

        row_chunks(lambda off: pltpu.sync_copy(
            agg_sh.at[pl.ds(off, RC)], agg_out.at[cid, pl.ds(off, RC)]))
        if with_cnt:
            off = pl.multiple_of(sid * CPT, 8)
            pltpu.sync_copy(cnt_sh.at[pl.ds(off, CPT)],
                            agg_out.at[cid, pl.ds(N + off, CPT)])

    cp = pltpu.CompilerParams()
    if "needs_layout_passes" in pltpu.CompilerParams.__dataclass_fields__:
        cp = dataclasses.replace(cp, needs_layout_passes=False)
    kern = pl.kernel(body, out_type=out_type, mesh=mesh,
                     scratch_types=scratch, compiler_params=cp)
    return kern(x, src, dst, zd)


BLK = 400


def _tc_dense(aggp, cntp, x, Wl, Wr, b, a):
    def body(aggp_ref, cntp_ref, x_ref, wl_ref, wr_ref, b_ref, a_ref, o_ref):
        s = aggp_ref[0] + aggp_ref[1]
        c = cntp_ref[0, :, 0:1] + cntp_ref[1, :, 0:1]
        agg = s / jnp.maximum(c, 1.0)
        h = (jnp.dot(agg, wl_ref[...], preferred_element_type=jnp.float32)
             + jnp.dot(x_ref[...], wr_ref[...], preferred_element_type=jnp.float32)
             + b_ref[...])
        o_ref[...] = jnp.where(h >= 0, h, a_ref[...] * h)

    return pl.pallas_call(
        body,
        grid=(N // BLK,),
        in_specs=[
            pl.BlockSpec((NC, BLK, D), lambda i: (0, i, 0)),
            pl.BlockSpec((NC, BLK, CNTW), lambda i: (0, i, 0)),
            pl.BlockSpec((BLK, D), lambda i: (i, 0)),
            pl.BlockSpec((D, D), lambda i: (0, 0)),
            pl.BlockSpec((D, D), lambda i: (0, 0)),
            pl.BlockSpec((1, D), lambda i: (0, 0)),
            pl.BlockSpec((1, D), lambda i: (0, 0)),
        ],
        out_specs=pl.BlockSpec((BLK, D), lambda i: (i, 0)),
        out_shape=jax.ShapeDtypeStruct((N, D), jnp.float32),
    )(aggp, cntp, x, Wl, Wr, b.reshape(1, D), a.reshape(1, D))


def kernel(x, edge_index, Wl1, Wr1, b1, a1, Wl2, Wr2, b2, a2):
    src = edge_index[0]
    dst = edge_index[1]
    zd = jnp.zeros((RC, D), jnp.float32)

    aggcnt = _sc_agg(x, src, dst, zd, with_cnt=True)
    agg1p = aggcnt[:, :N]
    cntp = aggcnt[:, N:].reshape(NC, NP, CNTW)[:, :N]
    h = _tc_dense(agg1p, cntp, x, Wl1, Wr1, b1, a1)
    agg2p = _sc_agg(h, src, dst, zd, with_cnt=False)
    return _tc_dense(agg2p, cntp, h, Wl2, Wr2, b2, a2)
